# Initial kernel scaffold; baseline (speedup 1.0000x reference)
import functools
import math

import jax
import jax.numpy as jnp
from jax import lax
from jax.experimental import pallas as pl
from jax.experimental.pallas import tpu as pltpu
from jax.experimental.pallas import tpu_sc as plsc

_B, _N, _E, _D, _H = 2, 10000, 160000, 128, 2
_NEG = 0.2
_EPS = 1e-16

_R = 500
_NT = _N // _R
_NS = 16
_NC = 2
_EPT = _E // _NS
_C = 80
_NCH = _EPT // _C
_FR = _N // _NS

_SQRT2 = math.sqrt(2.0)


def _epilogue(acch_ref, accd_ref, bias, ln_g, ln_b):
    m0 = acch_ref[0, 0]
    m1 = acch_ref[0, 1]
    d0 = accd_ref[0, 0][:, 0:1]
    d1 = accd_ref[0, 1][:, 0:1]
    x = 0.5 * (m0 / (d0 + _EPS) + m1 / (d1 + _EPS)) + bias[None, :]
    x = 0.5 * x * (1.0 + lax.erf(x / _SQRT2))
    mu = jnp.mean(x, axis=-1, keepdims=True)
    var = jnp.mean((x - mu) ** 2, axis=-1, keepdims=True)
    return (x - mu) * lax.rsqrt(var + 1e-5) * ln_g[None, :] + ln_b[None, :]


def _emit_tables(x, w_ref, asrc_ref, adst_ref, h_ref, sa_ref):
    h = jnp.dot(x, w_ref[...], preferred_element_type=jnp.float32)
    lane = lax.broadcasted_iota(jnp.int32, (_R, 16), 1)
    for k in range(_H):
        hk = h[:, k * _D:(k + 1) * _D]
        h_ref[0, k] = hk
        a_s = jnp.sum(hk * asrc_ref[k][None, :], axis=1)
        a_d = jnp.sum(hk * adst_ref[k][None, :], axis=1)
        sa_ref[0, k] = jnp.where(lane == 0, a_s[:, None],
                                 jnp.where(lane == 1, a_d[:, None], 0.0))


def _tc_first_body(x_ref, w_ref, asrc_ref, adst_ref, h_ref, sa_ref):
    _emit_tables(x_ref[0], w_ref, asrc_ref, adst_ref, h_ref, sa_ref)


def _tc_mid_body(acch_ref, accd_ref, b_ref, g_ref, be_ref,
                 w_ref, asrc_ref, adst_ref, h_ref, sa_ref):
    x = _epilogue(acch_ref, accd_ref, b_ref[...], g_ref[...], be_ref[...])
    _emit_tables(x, w_ref, asrc_ref, adst_ref, h_ref, sa_ref)


def _tc_final_body(acch_ref, accd_ref, b_ref, g_ref, be_ref, out_ref):
    out_ref[0] = _epilogue(acch_ref, accd_ref, b_ref[...], g_ref[...],
                           be_ref[...])


def _full(shape):
    return pl.BlockSpec(shape, lambda b, i: (0,) * len(shape))


_HS = pl.BlockSpec((1, _H, _R, _D), lambda b, i: (b, 0, i, 0))
_SAS = pl.BlockSpec((1, _H, _R, 16), lambda b, i: (b, 0, i, 0))
_TOUT = (jax.ShapeDtypeStruct((_B, _H, _N, _D), jnp.float32),
         jax.ShapeDtypeStruct((_B, _H, _N, 16), jnp.float32))

_tc_first = pl.pallas_call(
    _tc_first_body,
    grid=(_B, _NT),
    in_specs=[pl.BlockSpec((1, _R, _D), lambda b, i: (b, i, 0)),
              _full((_D, _H * _D)), _full((_H, _D)), _full((_H, _D))],
    out_specs=(_HS, _SAS),
    out_shape=_TOUT,
)

_tc_mid = pl.pallas_call(
    _tc_mid_body,
    grid=(_B, _NT),
    in_specs=[_HS, _SAS, _full((_D,)), _full((_D,)), _full((_D,)),
              _full((_D, _H * _D)), _full((_H, _D)), _full((_H, _D))],
    out_specs=(_HS, _SAS),
    out_shape=_TOUT,
)

_tc_final = pl.pallas_call(
    _tc_final_body,
    grid=(_B, _NT),
    in_specs=[_HS, _SAS, _full((_D,)), _full((_D,)), _full((_D,))],
    out_specs=pl.BlockSpec((1, _R, _D), lambda b, i: (b, i, 0)),
    out_shape=jax.ShapeDtypeStruct((_B, _N, _D), jnp.float32),
)


_mesh = plsc.VectorSubcoreMesh(core_axis_name="c", subcore_axis_name="s",
                               num_cores=_NC, num_subcores=_NS)


@functools.partial(
    pl.kernel,
    mesh=_mesh,
    out_type=(jax.ShapeDtypeStruct((_B, _H, _N, _D), jnp.float32),
              jax.ShapeDtypeStruct((_B, _H, _N, 16), jnp.float32)),
    scratch_types=dict(
        acch_s=pltpu.VMEM_SHARED((_N, _D), jnp.float32),
        accd_s=pltpu.VMEM_SHARED((_N, 16), jnp.float32),
        esrc=pltpu.VMEM((_C,), jnp.int32),
        edst=pltpu.VMEM((_C,), jnp.int32),
        gsrc=pltpu.VMEM((_C,), jnp.int32),
        gdst=pltpu.VMEM((_C,), jnp.int32),
        hrow=pltpu.VMEM((_C, _D), jnp.float32),
        sarow=pltpu.VMEM((_C, 16), jnp.float32),
        aarow=pltpu.VMEM((_C, 16), jnp.float32),
        wtail=pltpu.VMEM((_C, 16), jnp.float32),
        sem1=pltpu.SemaphoreType.DMA,
        sem2=pltpu.SemaphoreType.DMA,
        sem3=pltpu.SemaphoreType.DMA,
    ),
)
def _sc_edge(h_hbm, sa_hbm, edge_hbm, zh_hbm, zd_hbm, acch_hbm, accd_hbm,
             acch_s, accd_s, esrc, edst, gsrc, gdst, hrow, sarow, aarow,
             wtail, sem1, sem2, sem3):
    c = lax.axis_index("c")
    s = lax.axis_index("s")

    zero16f = jnp.zeros((16,), jnp.float32)
    z16 = jnp.zeros((16,), jnp.int32)
    o16 = jnp.ones((16,), jnp.int32)
    iota16 = lax.iota(jnp.int32, 16)

    for r in range(_C):
        wtail[r] = zero16f

    for b in range(_B):
        off = (b * _H + c) * _N

        rs = pl.ds(s * _FR, _FR)
        pltpu.sync_copy(zh_hbm, acch_s.at[rs])
        pltpu.sync_copy(zd_hbm, accd_s.at[rs])
        plsc.subcore_barrier()

        def chunk_body(t, carry):
            base = pl.multiple_of(s * _EPT + t * _C, 8)
            pltpu.sync_copy(edge_hbm.at[b, 0, pl.ds(base, _C)], esrc)
            pltpu.sync_copy(edge_hbm.at[b, 1, pl.ds(base, _C)], edst)
            for g in range(_C // 16):
                sl = pl.ds(16 * g, 16)
                gsrc[sl] = esrc[sl] + off
                gdst[sl] = edst[sl] + off
            d1 = pltpu.async_copy(h_hbm.at[gsrc], hrow, sem1)
            d2 = pltpu.async_copy(sa_hbm.at[gsrc], sarow, sem2)
            d3 = pltpu.async_copy(sa_hbm.at[gdst], aarow, sem3)
            d1.wait()
            d2.wait()
            d3.wait()
            for g in range(_C // 16):
                ids = iota16 + 16 * g
                a_s = plsc.load_gather(sarow, [ids, z16])
                a_d = plsc.load_gather(aarow, [ids, o16])
                al = a_s + a_d
                al = jnp.where(al >= 0, al, _NEG * al)
                w16 = jnp.exp(al)
                plsc.store_scatter(wtail, [ids, z16], w16)
                for l in range(16):
                    e = 16 * g + l
                    wspl = jnp.take(w16, jnp.full((16,), l, jnp.int32))
                    for j in range(_D // 16):
                        sl2 = pl.ds(16 * j, 16)
                        hrow[e, sl2] = hrow[e, sl2] * wspl
            pltpu.sync_copy(hrow, acch_s.at[edst], add=True)
            pltpu.sync_copy(wtail, accd_s.at[edst], add=True)
            return carry

        lax.fori_loop(0, _NCH, chunk_body, 0)
        plsc.subcore_barrier()

        pltpu.sync_copy(acch_s.at[rs], acch_hbm.at[b, c, rs])
        pltpu.sync_copy(accd_s.at[rs], accd_hbm.at[b, c, rs])


def kernel(v2, img_edge_index, W0, att_src0, att_dst0, b0,
           W1, att_src1, att_dst1, b1, ln_g, ln_b):
    edge = img_edge_index.astype(jnp.int32)
    zh = jnp.zeros((_FR, _D), jnp.float32)
    zd = jnp.zeros((_FR, 16), jnp.float32)

    h, sa = _tc_first(v2, W0, att_src0, att_dst0)
    acch, accd = _sc_edge(h.reshape(_B * _H * _N, _D),
                          sa.reshape(_B * _H * _N, 16), edge, zh, zd)
    h, sa = _tc_mid(acch, accd, b0, ln_g, ln_b, W1, att_src1, att_dst1)
    acch, accd = _sc_edge(h.reshape(_B * _H * _N, _D),
                          sa.reshape(_B * _H * _N, 16), edge, zh, zd)
    return _tc_final(acch, accd, b1, ln_g, ln_b)


# speedup vs baseline: 65.6319x; 65.6319x over previous
import functools
import math

import jax
import jax.numpy as jnp
from jax import lax
from jax.experimental import pallas as pl
from jax.experimental.pallas import tpu as pltpu
from jax.experimental.pallas import tpu_sc as plsc

_B, _N, _E, _D, _H = 2, 10000, 160000, 128, 2
_NEG = 0.2
_EPS = 1e-16

_R = 1000
_NT = _N // _R
_NS = 16
_NC = 2
_EPT = _E // _NS
_C = 80
_NCH = _EPT // _C
_FS = 624
_FZ = 640

_SQRT2 = math.sqrt(2.0)


def _epilogue(acch_ref, accd_ref, bias, ln_g, ln_b):
    m0 = acch_ref[0, 0]
    m1 = acch_ref[0, 1]
    d0 = accd_ref[0, 0][:, 0:1]
    d1 = accd_ref[0, 1][:, 0:1]
    x = 0.5 * (m0 / (d0 + _EPS) + m1 / (d1 + _EPS)) + bias[None, :]
    x = 0.5 * x * (1.0 + lax.erf(x / _SQRT2))
    mu = jnp.mean(x, axis=-1, keepdims=True)
    var = jnp.mean((x - mu) ** 2, axis=-1, keepdims=True)
    return (x - mu) * lax.rsqrt(var + 1e-5) * ln_g[None, :] + ln_b[None, :]


def _emit_tables(x, w_ref, asrc_ref, adst_ref, h_ref, sa_ref):
    h = jnp.dot(x, w_ref[...], preferred_element_type=jnp.float32)
    lane = lax.broadcasted_iota(jnp.int32, (_R, 16), 1)
    for k in range(_H):
        hk = h[:, k * _D:(k + 1) * _D]
        h_ref[0, k] = hk
        a_s = jnp.sum(hk * asrc_ref[k][None, :], axis=1)
        a_d = jnp.sum(hk * adst_ref[k][None, :], axis=1)
        sa_ref[0, k] = jnp.where(lane == 0, a_s[:, None],
                                 jnp.where(lane == 1, a_d[:, None], 0.0))


def _tc_first_body(x_ref, w_ref, asrc_ref, adst_ref, h_ref, sa_ref):
    _emit_tables(x_ref[0], w_ref, asrc_ref, adst_ref, h_ref, sa_ref)


def _tc_mid_body(acch_ref, accd_ref, b_ref, g_ref, be_ref,
                 w_ref, asrc_ref, adst_ref, h_ref, sa_ref):
    x = _epilogue(acch_ref, accd_ref, b_ref[...], g_ref[...], be_ref[...])
    _emit_tables(x, w_ref, asrc_ref, adst_ref, h_ref, sa_ref)


def _tc_final_body(acch_ref, accd_ref, b_ref, g_ref, be_ref, out_ref):
    out_ref[0] = _epilogue(acch_ref, accd_ref, b_ref[...], g_ref[...],
                           be_ref[...])


def _full(shape):
    return pl.BlockSpec(shape, lambda b, i: (0,) * len(shape))


_HS = pl.BlockSpec((1, _H, _R, _D), lambda b, i: (b, 0, i, 0))
_SAS = pl.BlockSpec((1, _H, _R, 16), lambda b, i: (b, 0, i, 0))
_TOUT = (jax.ShapeDtypeStruct((_B, _H, _N, _D), jnp.float32),
         jax.ShapeDtypeStruct((_B, _H, _N, 16), jnp.float32))

_tc_first = pl.pallas_call(
    _tc_first_body,
    grid=(_B, _NT),
    in_specs=[pl.BlockSpec((1, _R, _D), lambda b, i: (b, i, 0)),
              _full((_D, _H * _D)), _full((_H, _D)), _full((_H, _D))],
    out_specs=(_HS, _SAS),
    out_shape=_TOUT,
)

_tc_mid = pl.pallas_call(
    _tc_mid_body,
    grid=(_B, _NT),
    in_specs=[_HS, _SAS, _full((_D,)), _full((_D,)), _full((_D,)),
              _full((_D, _H * _D)), _full((_H, _D)), _full((_H, _D))],
    out_specs=(_HS, _SAS),
    out_shape=_TOUT,
)

_tc_final = pl.pallas_call(
    _tc_final_body,
    grid=(_B, _NT),
    in_specs=[_HS, _SAS, _full((_D,)), _full((_D,)), _full((_D,))],
    out_specs=pl.BlockSpec((1, _R, _D), lambda b, i: (b, i, 0)),
    out_shape=jax.ShapeDtypeStruct((_B, _N, _D), jnp.float32),
)


_mesh = plsc.VectorSubcoreMesh(core_axis_name="c", subcore_axis_name="s",
                               num_cores=_NC, num_subcores=_NS)


@functools.partial(
    pl.kernel,
    mesh=_mesh,
    compiler_params=pltpu.CompilerParams(needs_layout_passes=False,
                                         use_tc_tiling_on_sc=False),
    out_type=(jax.ShapeDtypeStruct((_B, _H, _N, _D), jnp.float32),
              jax.ShapeDtypeStruct((_B, _H, _N, 16), jnp.float32)),
    scratch_types=dict(
        acch_s=pltpu.VMEM_SHARED((_N, _D), jnp.float32),
        accd_s=pltpu.VMEM_SHARED((_N, 16), jnp.float32),
        esrc=pltpu.VMEM((_C,), jnp.int32),
        edst=pltpu.VMEM((_C,), jnp.int32),
        gsrc=pltpu.VMEM((_C,), jnp.int32),
        gdst=pltpu.VMEM((_C,), jnp.int32),
        hrow=pltpu.VMEM((_C, _D), jnp.float32),
        sarow=pltpu.VMEM((_C, 16), jnp.float32),
        aarow=pltpu.VMEM((_C, 16), jnp.float32),
        wtail=pltpu.VMEM((_C, 16), jnp.float32),
        sem1=pltpu.SemaphoreType.DMA,
        sem2=pltpu.SemaphoreType.DMA,
        sem3=pltpu.SemaphoreType.DMA,
    ),
)
def _sc_edge(h_hbm, sa_hbm, edge_hbm, zh_hbm, zd_hbm, acch_hbm, accd_hbm,
             acch_s, accd_s, esrc, edst, gsrc, gdst, hrow, sarow, aarow,
             wtail, sem1, sem2, sem3):
    c = lax.axis_index("c")
    s = lax.axis_index("s")

    zero16f = jnp.zeros((16,), jnp.float32)
    z16 = jnp.zeros((16,), jnp.int32)
    o16 = jnp.ones((16,), jnp.int32)
    iota16 = lax.iota(jnp.int32, 16)

    for r in range(_C):
        wtail[r] = zero16f

    for b in range(_B):
        off = (b * _H + c) * _N

        rs = pl.ds(pl.multiple_of(s * _FS, 8), _FZ)
        pltpu.sync_copy(zh_hbm, acch_s.at[rs])
        pltpu.sync_copy(zd_hbm, accd_s.at[rs])
        plsc.subcore_barrier()

        def chunk_body(t, carry):
            base = pl.multiple_of(s * _EPT + t * _C, 8)
            pltpu.sync_copy(edge_hbm.at[pl.ds(2 * b * _E + base, _C)], esrc)
            pltpu.sync_copy(edge_hbm.at[pl.ds((2 * b + 1) * _E + base, _C)],
                            edst)
            for g in range(_C // 16):
                sl = pl.ds(16 * g, 16)
                gsrc[sl] = esrc[sl] + off
                gdst[sl] = edst[sl] + off
            d1 = pltpu.async_copy(h_hbm.at[gsrc], hrow, sem1)
            d2 = pltpu.async_copy(sa_hbm.at[gsrc], sarow, sem2)
            d3 = pltpu.async_copy(sa_hbm.at[gdst], aarow, sem3)
            d1.wait()
            d2.wait()
            d3.wait()
            for g in range(_C // 16):
                ids = iota16 + 16 * g
                a_s = plsc.load_gather(sarow, [ids, z16])
                a_d = plsc.load_gather(aarow, [ids, o16])
                al = a_s + a_d
                al = jnp.where(al >= 0, al, _NEG * al)
                w16 = jnp.exp(al)
                plsc.store_scatter(wtail, [ids, z16], w16)
                for l in range(16):
                    e = 16 * g + l
                    wspl = jnp.take(w16, jnp.full((16,), l, jnp.int32))
                    for j in range(_D // 16):
                        sl2 = pl.ds(16 * j, 16)
                        hrow[e, sl2] = hrow[e, sl2] * wspl
            pltpu.sync_copy(hrow, acch_s.at[edst], add=True)
            pltpu.sync_copy(wtail, accd_s.at[edst], add=True)
            return carry

        lax.fori_loop(0, _NCH, chunk_body, 0)
        plsc.subcore_barrier()

        pltpu.sync_copy(acch_s.at[rs], acch_hbm.at[b, c, rs])
        pltpu.sync_copy(accd_s.at[rs], accd_hbm.at[b, c, rs])


def kernel(v2, img_edge_index, W0, att_src0, att_dst0, b0,
           W1, att_src1, att_dst1, b1, ln_g, ln_b):
    edge = img_edge_index.astype(jnp.int32).reshape(_B * 2 * _E)
    zh = jnp.zeros((_FZ, _D), jnp.float32)
    zd = jnp.zeros((_FZ, 16), jnp.float32)

    h, sa = _tc_first(v2, W0, att_src0, att_dst0)
    acch, accd = _sc_edge(h.reshape(_B * _H * _N, _D),
                          sa.reshape(_B * _H * _N, 16), edge, zh, zd)
    h, sa = _tc_mid(acch, accd, b0, ln_g, ln_b, W1, att_src1, att_dst1)
    acch, accd = _sc_edge(h.reshape(_B * _H * _N, _D),
                          sa.reshape(_B * _H * _N, 16), edge, zh, zd)
    return _tc_final(acch, accd, b1, ln_g, ln_b)


# speedup vs baseline: 80.0863x; 1.2202x over previous
import functools
import math

import jax
import jax.numpy as jnp
from jax import lax
from jax.experimental import pallas as pl
from jax.experimental.pallas import tpu as pltpu
from jax.experimental.pallas import tpu_sc as plsc

_B, _N, _E, _D, _H = 2, 10000, 160000, 128, 2
_NEG = 0.2
_EPS = 1e-16

_R = 1000
_NT = _N // _R
_NS = 16
_NC = 2
_EPT = _E // _NS
_C = 80
_NCH = _EPT // _C
_FS = 624
_FZ = 640

_SQRT2 = math.sqrt(2.0)


def _epilogue(acch_ref, accd_ref, bias, ln_g, ln_b):
    m0 = acch_ref[0, 0]
    m1 = acch_ref[0, 1]
    d0 = accd_ref[0, 0][:, 0:1]
    d1 = accd_ref[0, 1][:, 0:1]
    x = 0.5 * (m0 / (d0 + _EPS) + m1 / (d1 + _EPS)) + bias[None, :]
    x = 0.5 * x * (1.0 + lax.erf(x / _SQRT2))
    mu = jnp.mean(x, axis=-1, keepdims=True)
    var = jnp.mean((x - mu) ** 2, axis=-1, keepdims=True)
    return (x - mu) * lax.rsqrt(var + 1e-5) * ln_g[None, :] + ln_b[None, :]


def _emit_tables(x, w_ref, asrc_ref, adst_ref, h_ref, sa_ref):
    h = jnp.dot(x, w_ref[...], preferred_element_type=jnp.float32)
    lane = lax.broadcasted_iota(jnp.int32, (_R, 16), 1)
    for k in range(_H):
        hk = h[:, k * _D:(k + 1) * _D]
        h_ref[0, k] = hk
        a_s = jnp.sum(hk * asrc_ref[k][None, :], axis=1)
        a_d = jnp.sum(hk * adst_ref[k][None, :], axis=1)
        sa_ref[0, k] = jnp.where(lane == 0, a_s[:, None],
                                 jnp.where(lane == 1, a_d[:, None], 0.0))


def _tc_first_body(x_ref, w_ref, asrc_ref, adst_ref, h_ref, sa_ref):
    _emit_tables(x_ref[0], w_ref, asrc_ref, adst_ref, h_ref, sa_ref)


def _tc_mid_body(acch_ref, accd_ref, b_ref, g_ref, be_ref,
                 w_ref, asrc_ref, adst_ref, h_ref, sa_ref):
    x = _epilogue(acch_ref, accd_ref, b_ref[...], g_ref[...], be_ref[...])
    _emit_tables(x, w_ref, asrc_ref, adst_ref, h_ref, sa_ref)


def _tc_final_body(acch_ref, accd_ref, b_ref, g_ref, be_ref, out_ref):
    out_ref[0] = _epilogue(acch_ref, accd_ref, b_ref[...], g_ref[...],
                           be_ref[...])


def _full(shape):
    return pl.BlockSpec(shape, lambda b, i: (0,) * len(shape))


_HS = pl.BlockSpec((1, _H, _R, _D), lambda b, i: (b, 0, i, 0))
_SAS = pl.BlockSpec((1, _H, _R, 16), lambda b, i: (b, 0, i, 0))
_TOUT = (jax.ShapeDtypeStruct((_B, _H, _N, _D), jnp.float32),
         jax.ShapeDtypeStruct((_B, _H, _N, 16), jnp.float32))

_tc_first = pl.pallas_call(
    _tc_first_body,
    grid=(_B, _NT),
    in_specs=[pl.BlockSpec((1, _R, _D), lambda b, i: (b, i, 0)),
              _full((_D, _H * _D)), _full((_H, _D)), _full((_H, _D))],
    out_specs=(_HS, _SAS),
    out_shape=_TOUT,
)

_tc_mid = pl.pallas_call(
    _tc_mid_body,
    grid=(_B, _NT),
    in_specs=[_HS, _SAS, _full((_D,)), _full((_D,)), _full((_D,)),
              _full((_D, _H * _D)), _full((_H, _D)), _full((_H, _D))],
    out_specs=(_HS, _SAS),
    out_shape=_TOUT,
)

_tc_final = pl.pallas_call(
    _tc_final_body,
    grid=(_B, _NT),
    in_specs=[_HS, _SAS, _full((_D,)), _full((_D,)), _full((_D,))],
    out_specs=pl.BlockSpec((1, _R, _D), lambda b, i: (b, i, 0)),
    out_shape=jax.ShapeDtypeStruct((_B, _N, _D), jnp.float32),
)


_mesh = plsc.VectorSubcoreMesh(core_axis_name="c", subcore_axis_name="s",
                               num_cores=_NC, num_subcores=_NS)


@functools.partial(
    pl.kernel,
    mesh=_mesh,
    compiler_params=pltpu.CompilerParams(needs_layout_passes=False,
                                         use_tc_tiling_on_sc=False),
    out_type=(jax.ShapeDtypeStruct((_B, _H, _N, _D), jnp.float32),
              jax.ShapeDtypeStruct((_B, _H, _N, 16), jnp.float32)),
    scratch_types=dict(
        acch_s=pltpu.VMEM_SHARED((_N, _D), jnp.float32),
        accd_s=pltpu.VMEM_SHARED((_N, 16), jnp.float32),
        esrc=[pltpu.VMEM((_C,), jnp.int32)] * 2,
        edst=[pltpu.VMEM((_C,), jnp.int32)] * 2,
        gsrc=[pltpu.VMEM((_C,), jnp.int32)] * 2,
        gdst=[pltpu.VMEM((_C,), jnp.int32)] * 2,
        hrow=[pltpu.VMEM((_C, _D), jnp.float32)] * 2,
        sarow=[pltpu.VMEM((_C, 16), jnp.float32)] * 2,
        aarow=[pltpu.VMEM((_C, 16), jnp.float32)] * 2,
        wtail=[pltpu.VMEM((_C, 16), jnp.float32)] * 2,
        gsem=[pltpu.SemaphoreType.DMA] * 2,
    ),
)
def _sc_edge(h_hbm, sa_hbm, edge_hbm, zh_hbm, zd_hbm, acch_hbm, accd_hbm,
             acch_s, accd_s, esrc, edst, gsrc, gdst, hrow, sarow, aarow,
             wtail, gsem):
    c = lax.axis_index("c")
    s = lax.axis_index("s")

    zero16f = jnp.zeros((16,), jnp.float32)
    z16 = jnp.zeros((16,), jnp.int32)
    o16 = jnp.ones((16,), jnp.int32)
    iota16 = lax.iota(jnp.int32, 16)

    for i in range(2):
        for r in range(_C):
            wtail[i][r] = zero16f

    def fire(b, t, i):
        off = (b * _H + c) * _N
        base = pl.multiple_of(s * _EPT + t * _C, 8)
        pltpu.sync_copy(edge_hbm.at[pl.ds(2 * b * _E + base, _C)], esrc[i])
        pltpu.sync_copy(edge_hbm.at[pl.ds((2 * b + 1) * _E + base, _C)],
                        edst[i])
        for g in range(_C // 16):
            sl = pl.ds(16 * g, 16)
            gsrc[i][sl] = esrc[i][sl] + off
            gdst[i][sl] = edst[i][sl] + off
        pltpu.async_copy(h_hbm.at[gsrc[i]], hrow[i], gsem[i])
        pltpu.async_copy(sa_hbm.at[gsrc[i]], sarow[i], gsem[i])
        pltpu.async_copy(sa_hbm.at[gdst[i]], aarow[i], gsem[i])

    def finish(i):
        pltpu.make_async_copy(h_hbm.at[gsrc[i]], hrow[i], gsem[i]).wait()
        pltpu.make_async_copy(sa_hbm.at[gsrc[i]], sarow[i], gsem[i]).wait()
        pltpu.make_async_copy(sa_hbm.at[gdst[i]], aarow[i], gsem[i]).wait()
        for g in range(_C // 16):
            ids = iota16 + 16 * g
            a_s = plsc.load_gather(sarow[i], [ids, z16])
            a_d = plsc.load_gather(aarow[i], [ids, o16])
            al = a_s + a_d
            al = jnp.where(al >= 0, al, _NEG * al)
            w16 = jnp.exp(al)
            plsc.store_scatter(wtail[i], [ids, z16], w16)
            for l in range(16):
                e = 16 * g + l
                wspl = jnp.take(w16, jnp.full((16,), l, jnp.int32))
                for j in range(_D // 16):
                    sl2 = pl.ds(16 * j, 16)
                    hrow[i][e, sl2] = hrow[i][e, sl2] * wspl
        pltpu.sync_copy(hrow[i], acch_s.at[edst[i]], add=True)
        pltpu.sync_copy(wtail[i], accd_s.at[edst[i]], add=True)

    for b in range(_B):
        rs = pl.ds(pl.multiple_of(s * _FS, 8), _FZ)
        pltpu.sync_copy(zh_hbm, acch_s.at[rs])
        pltpu.sync_copy(zd_hbm, accd_s.at[rs])
        plsc.subcore_barrier()

        fire(b, 0, 0)

        def pair_body(p, carry):
            fire(b, 2 * p + 1, 1)
            finish(0)
            fire(b, 2 * p + 2, 0)
            finish(1)
            return carry

        lax.fori_loop(0, _NCH // 2, pair_body, 0)
        finish(0)
        plsc.subcore_barrier()

        pltpu.sync_copy(acch_s.at[rs], acch_hbm.at[b, c, rs])
        pltpu.sync_copy(accd_s.at[rs], accd_hbm.at[b, c, rs])


def kernel(v2, img_edge_index, W0, att_src0, att_dst0, b0,
           W1, att_src1, att_dst1, b1, ln_g, ln_b):
    edge = img_edge_index.astype(jnp.int32).reshape(_B * 2 * _E)
    zh = jnp.zeros((_FZ, _D), jnp.float32)
    zd = jnp.zeros((_FZ, 16), jnp.float32)

    h, sa = _tc_first(v2, W0, att_src0, att_dst0)
    acch, accd = _sc_edge(h.reshape(_B * _H * _N, _D),
                          sa.reshape(_B * _H * _N, 16), edge, zh, zd)
    h, sa = _tc_mid(acch, accd, b0, ln_g, ln_b, W1, att_src1, att_dst1)
    acch, accd = _sc_edge(h.reshape(_B * _H * _N, _D),
                          sa.reshape(_B * _H * _N, 16), edge, zh, zd)
    return _tc_final(acch, accd, b1, ln_g, ln_b)


# speedup vs baseline: 86.0683x; 1.0747x over previous
import functools
import math

import jax
import jax.numpy as jnp
from jax import lax
from jax.experimental import pallas as pl
from jax.experimental.pallas import tpu as pltpu
from jax.experimental.pallas import tpu_sc as plsc

_B, _N, _E, _D, _H = 2, 10000, 160000, 128, 2
_NEG = 0.2
_EPS = 1e-16

_R = 1000
_NT = _N // _R
_NS = 16
_NC = 2
_EPT = _E // _NS
_C = 80
_NCH = _EPT // _C
_NP = (_NCH - 1) // 2
_FS = 624
_FZ = 640

_SQRT2 = math.sqrt(2.0)


def _epilogue(acch_ref, accd_ref, bias, ln_g, ln_b):
    m0 = acch_ref[0, 0]
    m1 = acch_ref[0, 1]
    d0 = accd_ref[0, 0][:, 0:1]
    d1 = accd_ref[0, 1][:, 0:1]
    x = 0.5 * (m0 / (d0 + _EPS) + m1 / (d1 + _EPS)) + bias[None, :]
    x = 0.5 * x * (1.0 + lax.erf(x / _SQRT2))
    mu = jnp.mean(x, axis=-1, keepdims=True)
    var = jnp.mean((x - mu) ** 2, axis=-1, keepdims=True)
    return (x - mu) * lax.rsqrt(var + 1e-5) * ln_g[None, :] + ln_b[None, :]


def _emit_tables(x, w_ref, asrc_ref, adst_ref, h_ref, sa_ref):
    h = jnp.dot(x, w_ref[...], preferred_element_type=jnp.float32)
    lane = lax.broadcasted_iota(jnp.int32, (_R, 16), 1)
    for k in range(_H):
        hk = h[:, k * _D:(k + 1) * _D]
        h_ref[k, 0] = hk
        a_s = jnp.sum(hk * asrc_ref[k][None, :], axis=1)
        a_d = jnp.sum(hk * adst_ref[k][None, :], axis=1)
        sa_ref[k, 0] = jnp.where(lane == 0, a_s[:, None],
                                 jnp.where(lane == 1, a_d[:, None], 0.0))


def _tc_first_body(x_ref, w_ref, asrc_ref, adst_ref, h_ref, sa_ref):
    _emit_tables(x_ref[0], w_ref, asrc_ref, adst_ref, h_ref, sa_ref)


def _tc_mid_body(acch_ref, accd_ref, b_ref, g_ref, be_ref,
                 w_ref, asrc_ref, adst_ref, h_ref, sa_ref):
    x = _epilogue(acch_ref, accd_ref, b_ref[...], g_ref[...], be_ref[...])
    _emit_tables(x, w_ref, asrc_ref, adst_ref, h_ref, sa_ref)


def _tc_final_body(acch_ref, accd_ref, b_ref, g_ref, be_ref, out_ref):
    out_ref[0] = _epilogue(acch_ref, accd_ref, b_ref[...], g_ref[...],
                           be_ref[...])


def _full(shape):
    return pl.BlockSpec(shape, lambda b, i: (0,) * len(shape))


_ACCHS = pl.BlockSpec((1, _H, _R, _D), lambda b, i: (b, 0, i, 0))
_ACCDS = pl.BlockSpec((1, _H, _R, 16), lambda b, i: (b, 0, i, 0))
_HS = pl.BlockSpec((_H, 1, _R, _D), lambda b, i: (0, b, i, 0))
_SAS = pl.BlockSpec((_H, 1, _R, 16), lambda b, i: (0, b, i, 0))
_TOUT = (jax.ShapeDtypeStruct((_H, _B, _N, _D), jnp.float32),
         jax.ShapeDtypeStruct((_H, _B, _N, 16), jnp.float32))

_tc_first = pl.pallas_call(
    _tc_first_body,
    grid=(_B, _NT),
    in_specs=[pl.BlockSpec((1, _R, _D), lambda b, i: (b, i, 0)),
              _full((_D, _H * _D)), _full((_H, _D)), _full((_H, _D))],
    out_specs=(_HS, _SAS),
    out_shape=_TOUT,
)

_tc_mid = pl.pallas_call(
    _tc_mid_body,
    grid=(_B, _NT),
    in_specs=[_ACCHS, _ACCDS, _full((_D,)), _full((_D,)), _full((_D,)),
              _full((_D, _H * _D)), _full((_H, _D)), _full((_H, _D))],
    out_specs=(_HS, _SAS),
    out_shape=_TOUT,
)

_tc_final = pl.pallas_call(
    _tc_final_body,
    grid=(_B, _NT),
    in_specs=[_ACCHS, _ACCDS, _full((_D,)), _full((_D,)), _full((_D,))],
    out_specs=pl.BlockSpec((1, _R, _D), lambda b, i: (b, i, 0)),
    out_shape=jax.ShapeDtypeStruct((_B, _N, _D), jnp.float32),
)


_mesh = plsc.VectorSubcoreMesh(core_axis_name="c", subcore_axis_name="s",
                               num_cores=_NC, num_subcores=_NS)


@functools.partial(
    pl.kernel,
    mesh=_mesh,
    compiler_params=pltpu.CompilerParams(needs_layout_passes=False,
                                         use_tc_tiling_on_sc=False),
    out_type=(jax.ShapeDtypeStruct((_B, _H, _N, _D), jnp.float32),
              jax.ShapeDtypeStruct((_B, _H, _N, 16), jnp.float32)),
    scratch_types=dict(
        acch_s=pltpu.VMEM_SHARED((_N, _D), jnp.float32),
        accd_s=pltpu.VMEM_SHARED((_N, 16), jnp.float32),
        esrc=[pltpu.VMEM((_C,), jnp.int32)] * 2,
        egdst=[pltpu.VMEM((_C,), jnp.int32)] * 2,
        ldst=[pltpu.VMEM((_C,), jnp.int32)] * 2,
        hrow=[pltpu.VMEM((_C, _D), jnp.float32)] * 2,
        sarow=[pltpu.VMEM((_C, 16), jnp.float32)] * 2,
        aarow=[pltpu.VMEM((_C, 16), jnp.float32)] * 2,
        wtail=[pltpu.VMEM((_C, 16), jnp.float32)] * 2,
        isem=[pltpu.SemaphoreType.DMA] * 2,
        gsem=[pltpu.SemaphoreType.DMA] * 2,
        ssem=[pltpu.SemaphoreType.DMA] * 2,
    ),
)
def _sc_edge(h_hbm, sa_hbm, gsi_hbm, gdi_hbm, zh_hbm, zd_hbm,
             acch_hbm, accd_hbm, acch_s, accd_s, esrc, egdst, ldst,
             hrow, sarow, aarow, wtail, isem, gsem, ssem):
    c = lax.axis_index("c")
    s = lax.axis_index("s")
    cbase = c * (_B * _E)

    zero16f = jnp.zeros((16,), jnp.float32)
    z16 = jnp.zeros((16,), jnp.int32)
    o16 = jnp.ones((16,), jnp.int32)
    iota16 = lax.iota(jnp.int32, 16)

    for i in range(2):
        for r in range(_C):
            wtail[i][r] = zero16f

    def idx_base(b, t):
        return pl.multiple_of(cbase + b * _E + s * _EPT + t * _C, 8)

    def fire_idx(b, t, i):
        bs = idx_base(b, t)
        pltpu.async_copy(gsi_hbm.at[pl.ds(bs, _C)], esrc[i], isem[i])
        pltpu.async_copy(gdi_hbm.at[pl.ds(bs, _C)], egdst[i], isem[i])

    def fire_gath(b, t, i, offn):
        bs = idx_base(b, t)
        pltpu.make_async_copy(gsi_hbm.at[pl.ds(bs, _C)], esrc[i],
                              isem[i]).wait()
        pltpu.make_async_copy(gdi_hbm.at[pl.ds(bs, _C)], egdst[i],
                              isem[i]).wait()
        for g in range(_C // 16):
            sl = pl.ds(16 * g, 16)
            ldst[i][sl] = egdst[i][sl] - offn
        pltpu.async_copy(h_hbm.at[esrc[i]], hrow[i], gsem[i])
        pltpu.async_copy(sa_hbm.at[esrc[i]], sarow[i], gsem[i])
        pltpu.async_copy(sa_hbm.at[egdst[i]], aarow[i], gsem[i])

    def wait_gath(i):
        pltpu.make_async_copy(h_hbm.at[esrc[i]], hrow[i], gsem[i]).wait()
        pltpu.make_async_copy(sa_hbm.at[esrc[i]], sarow[i], gsem[i]).wait()
        pltpu.make_async_copy(sa_hbm.at[egdst[i]], aarow[i], gsem[i]).wait()

    def compute(i):
        for g in range(_C // 16):
            ids = iota16 + 16 * g
            a_s = plsc.load_gather(sarow[i], [ids, z16])
            a_d = plsc.load_gather(aarow[i], [ids, o16])
            al = a_s + a_d
            al = jnp.where(al >= 0, al, _NEG * al)
            w16 = jnp.exp(al)
            plsc.store_scatter(wtail[i], [ids, z16], w16)
            for l in range(16):
                e = 16 * g + l
                wspl = jnp.take(w16, jnp.full((16,), l, jnp.int32))
                for j in range(_D // 16):
                    sl2 = pl.ds(16 * j, 16)
                    hrow[i][e, sl2] = hrow[i][e, sl2] * wspl

    def fire_scat(i):
        pltpu.async_copy(hrow[i], acch_s.at[ldst[i]], ssem[i], add=True)
        pltpu.async_copy(wtail[i], accd_s.at[ldst[i]], ssem[i], add=True)

    def wait_scat(i):
        pltpu.make_async_copy(hrow[i], acch_s.at[ldst[i]], ssem[i]).wait()
        pltpu.make_async_copy(wtail[i], accd_s.at[ldst[i]], ssem[i]).wait()

    for b in range(_B):
        offn = (c * _B + b) * _N

        rs = pl.ds(pl.multiple_of(s * _FS, 8), _FZ)
        pltpu.sync_copy(zh_hbm, acch_s.at[rs])
        pltpu.sync_copy(zd_hbm, accd_s.at[rs])
        plsc.subcore_barrier()

        fire_idx(b, 0, 0)
        fire_gath(b, 0, 0, offn)
        wait_gath(0)
        compute(0)
        fire_scat(0)
        fire_idx(b, 1, 1)
        fire_gath(b, 1, 1, offn)
        fire_idx(b, 2, 0)

        def macro_body(p, carry):
            for k in range(2):
                t = 2 * p + 1 + k
                i = (1 + k) % 2
                o = k % 2
                wait_gath(i)
                compute(i)
                fire_scat(i)
                wait_scat(o)
                if k == 0:
                    fire_gath(b, t + 1, o, offn)
                else:
                    @pl.when(p < _NP - 1)
                    def _():
                        fire_gath(b, t + 1, o, offn)

                @pl.when(p < _NP - 1)
                def _():
                    fire_idx(b, t + 2, i)
            return carry

        lax.fori_loop(0, _NP, macro_body, 0)
        wait_scat(0)
        plsc.subcore_barrier()

        pltpu.sync_copy(acch_s.at[rs], acch_hbm.at[b, c, rs])
        pltpu.sync_copy(accd_s.at[rs], accd_hbm.at[b, c, rs])


def kernel(v2, img_edge_index, W0, att_src0, att_dst0, b0,
           W1, att_src1, att_dst1, b1, ln_g, ln_b):
    edge = img_edge_index.astype(jnp.int32)
    offs = ((lax.broadcasted_iota(jnp.int32, (_H, _B), 0) * _B
             + lax.broadcasted_iota(jnp.int32, (_H, _B), 1)) * _N)
    gsi = (edge[None, :, 0, :] + offs[:, :, None]).reshape(_H * _B * _E)
    gdi = (edge[None, :, 1, :] + offs[:, :, None]).reshape(_H * _B * _E)
    zh = jnp.zeros((_FZ, _D), jnp.float32)
    zd = jnp.zeros((_FZ, 16), jnp.float32)

    h, sa = _tc_first(v2, W0, att_src0, att_dst0)
    acch, accd = _sc_edge(h.reshape(_H * _B * _N, _D),
                          sa.reshape(_H * _B * _N, 16), gsi, gdi, zh, zd)
    h, sa = _tc_mid(acch, accd, b0, ln_g, ln_b, W1, att_src1, att_dst1)
    acch, accd = _sc_edge(h.reshape(_H * _B * _N, _D),
                          sa.reshape(_H * _B * _N, 16), gsi, gdi, zh, zd)
    return _tc_final(acch, accd, b1, ln_g, ln_b)
